# baseline (speedup 1.0000x reference)
import functools
import jax
import jax.numpy as jnp
from jax import lax
from jax.experimental import pallas as pl
from jax.experimental.pallas import tpu as pltpu
from jax.experimental.pallas import tpu_sc as plsc

VOCAB = 1000
B, T = 1024, 50
VP = 1024
NVB = VOCAB // 8

NC = 2
NS = 16
NW = NC * NS
KMAX = -(-NVB // NW)

_mesh = plsc.VectorSubcoreMesh(
    core_axis_name="c", subcore_axis_name="s", num_cores=NC, num_subcores=NS
)


@functools.partial(
    pl.kernel,
    out_type=jax.ShapeDtypeStruct((T, VOCAB, B), jnp.float32),
    mesh=_mesh,
    scratch_types=[
        pltpu.VMEM((T * B,), jnp.int32),
        pltpu.VMEM((8 * VP,), jnp.float32),
        pltpu.VMEM((8, B), jnp.float32),
        pltpu.VMEM((8, B), jnp.float32),
        pltpu.SemaphoreType.DMA,
        pltpu.SemaphoreType.DMA,
    ],
    compiler_params=pltpu.CompilerParams(use_tc_tiling_on_sc=False, needs_layout_passes=False),
)
def _sc_tgather(tableT_hbm, idxT_hbm, out_hbm, idx_v, ts, asm0, asm1, ws0, ws1):
    wid = lax.axis_index("s") * NC + lax.axis_index("c")
    pltpu.sync_copy(idxT_hbm, idx_v)

    asms = (asm0, asm1)
    wsems = (ws0, ws1)

    def wdesc(p, t, v0):
        return pltpu.make_async_copy(
            asms[p], out_hbm.at[t].at[pl.ds(v0, 8), :], wsems[p]
        )

    @pl.loop(0, KMAX)
    def _round(k):
        vb = k * NW + wid

        @pl.when(vb < NVB)
        def _():
            v0 = pl.multiple_of(vb * 8, 8)
            pltpu.sync_copy(
                tableT_hbm.at[pl.ds(pl.multiple_of(v0 * VP, 8), 8 * VP)], ts
            )

            @pl.loop(0, T, step=2)
            def _t2(t0):
                for p in range(2):
                    t = t0 + p

                    @pl.when(t >= 2)
                    def _reuse():
                        wdesc(p, t, v0).wait()

                    asm = asms[p]
                    for bc in range(B // 16):
                        iv = idx_v[pl.ds(pl.multiple_of(t * B, 16) + bc * 16, 16)]
                        for v in range(8):
                            g = plsc.load_gather(ts, [iv + v * VP])
                            asm[v, pl.ds(bc * 16, 16)] = g
                    wdesc(p, t, v0).start()

            wdesc(0, 0, v0).wait()
            wdesc(1, 1, v0).wait()


def kernel(idx, table):
    idxT = idx.T.astype(jnp.int32).reshape(-1)
    tableT = jnp.pad(table.T, ((0, 0), (0, VP - VOCAB))).reshape(-1)
    a2 = _sc_tgather(tableT, idxT)
    return a2.transpose(2, 0, 1)


# speedup vs baseline: 1.4768x; 1.4768x over previous
import functools
import jax
import jax.numpy as jnp
from jax import lax
from jax.experimental import pallas as pl
from jax.experimental.pallas import tpu as pltpu
from jax.experimental.pallas import tpu_sc as plsc

VOCAB = 1000
B, T = 1024, 50
VP = 1024
NVB = VOCAB // 8

NC = 2
NS = 16
NW = NC * NS
KMAX = -(-NVB // NW)

_mesh = plsc.VectorSubcoreMesh(
    core_axis_name="c", subcore_axis_name="s", num_cores=NC, num_subcores=NS
)


@functools.partial(
    pl.kernel,
    out_type=jax.ShapeDtypeStruct((T, VOCAB, B), jnp.float32),
    mesh=_mesh,
    scratch_types=[
        pltpu.VMEM((T * B,), jnp.int32),
        pltpu.VMEM((8 * VP,), jnp.float32),
        pltpu.VMEM((8, B), jnp.float32),
        pltpu.VMEM((8, B), jnp.float32),
        pltpu.SemaphoreType.DMA,
        pltpu.SemaphoreType.DMA,
    ],
    compiler_params=pltpu.CompilerParams(use_tc_tiling_on_sc=False, needs_layout_passes=False),
)
def _sc_tgather(tableT_hbm, idxT_hbm, out_hbm, idx_v, ts, asm0, asm1, ws0, ws1):
    wid = lax.axis_index("s") * NC + lax.axis_index("c")
    pltpu.sync_copy(idxT_hbm, idx_v)

    asms = (asm0, asm1)
    wsems = (ws0, ws1)

    def wdesc(p, t, v0):
        return pltpu.make_async_copy(
            asms[p], out_hbm.at[t].at[pl.ds(v0, 8), :], wsems[p]
        )

    @pl.loop(0, KMAX)
    def _round(k):
        vb = k * NW + wid

        @pl.when(vb < NVB)
        def _():
            v0 = pl.multiple_of(vb * 8, 8)
            pltpu.sync_copy(
                tableT_hbm.at[pl.ds(pl.multiple_of(v0 * VP, 8), 8 * VP)], ts
            )

            @pl.loop(0, T, step=2)
            def _t2(t0):
                for p in range(2):
                    t = t0 + p

                    @pl.when(t >= 2)
                    def _reuse():
                        wdesc(p, t, v0).wait()

                    asm = asms[p]
                    for bc0 in range(0, B // 16, 2):
                        ivs = [
                            idx_v[
                                pl.ds(
                                    pl.multiple_of(t * B, 16) + (bc0 + u) * 16,
                                    16,
                                )
                            ]
                            for u in range(2)
                        ]
                        gs = [
                            [plsc.load_gather(ts, [ivs[u] + v * VP]) for v in range(8)]
                            for u in range(2)
                        ]
                        for u in range(2):
                            for v in range(8):
                                asm[v, pl.ds((bc0 + u) * 16, 16)] = gs[u][v]
                    wdesc(p, t, v0).start()

            wdesc(0, 0, v0).wait()
            wdesc(1, 1, v0).wait()


def kernel(idx, table):
    idxT = idx.T.astype(jnp.int32).reshape(-1)
    tableT = jnp.pad(table.T, ((0, 0), (0, VP - VOCAB))).reshape(-1)
    a2 = _sc_tgather(tableT, idxT)
    return a2.transpose(2, 0, 1)


# speedup vs baseline: 1.5362x; 1.0402x over previous
import functools
import jax
import jax.numpy as jnp
from jax import lax
from jax.experimental import pallas as pl
from jax.experimental.pallas import tpu as pltpu
from jax.experimental.pallas import tpu_sc as plsc

VOCAB = 1000
B, T = 1024, 50
VP = 1024
NVB = VOCAB // 8

NC = 2
NS = 16
NW = NC * NS
KMAX = -(-NVB // NW)

_mesh = plsc.VectorSubcoreMesh(
    core_axis_name="c", subcore_axis_name="s", num_cores=NC, num_subcores=NS
)


@functools.partial(
    pl.kernel,
    out_type=jax.ShapeDtypeStruct((T, VOCAB, B), jnp.float32),
    mesh=_mesh,
    scratch_types=[
        pltpu.VMEM((T * B,), jnp.int32),
        pltpu.VMEM((8 * VP,), jnp.float32),
        pltpu.VMEM((8, B), jnp.float32),
        pltpu.VMEM((8, B), jnp.float32),
        pltpu.SemaphoreType.DMA,
        pltpu.SemaphoreType.DMA,
    ],
    compiler_params=pltpu.CompilerParams(use_tc_tiling_on_sc=False, needs_layout_passes=False),
)
def _sc_tgather(tableT_hbm, idxT_hbm, out_hbm, idx_v, ts, asm0, asm1, ws0, ws1):
    wid = lax.axis_index("s") * NC + lax.axis_index("c")
    pltpu.sync_copy(idxT_hbm, idx_v)

    asms = (asm0, asm1)
    wsems = (ws0, ws1)

    def wdesc(p, t, v0):
        return pltpu.make_async_copy(
            asms[p], out_hbm.at[t].at[pl.ds(v0, 8), :], wsems[p]
        )

    @pl.loop(0, KMAX)
    def _round(k):
        vb = k * NW + wid

        @pl.when(vb < NVB)
        def _():
            v0 = pl.multiple_of(vb * 8, 8)
            pltpu.sync_copy(
                tableT_hbm.at[pl.ds(pl.multiple_of(v0 * VP, 8), 8 * VP)], ts
            )

            @pl.loop(0, T, step=2)
            def _t2(t0):
                for p in range(2):
                    t = t0 + p

                    @pl.when(t >= 2)
                    def _reuse():
                        wdesc(p, t, v0).wait()

                    asm = asms[p]
                    for bc0 in range(0, B // 16, 4):
                        ivs = [
                            idx_v[
                                pl.ds(
                                    pl.multiple_of(t * B, 16) + (bc0 + u) * 16,
                                    16,
                                )
                            ]
                            for u in range(4)
                        ]
                        gs = [
                            [plsc.load_gather(ts, [ivs[u] + v * VP]) for v in range(8)]
                            for u in range(4)
                        ]
                        for u in range(4):
                            for v in range(8):
                                asm[v, pl.ds((bc0 + u) * 16, 16)] = gs[u][v]
                    wdesc(p, t, v0).start()

            wdesc(0, 0, v0).wait()
            wdesc(1, 1, v0).wait()


def kernel(idx, table):
    idxT = idx.T.astype(jnp.int32).reshape(-1)
    tableT = jnp.pad(table.T, ((0, 0), (0, VP - VOCAB))).reshape(-1)
    a2 = _sc_tgather(tableT, idxT)
    return a2.transpose(2, 0, 1)
